# baseline (speedup 1.0000x reference)
import functools

import jax
import jax.numpy as jnp
from jax import lax
from jax.experimental import pallas as pl
from jax.experimental.pallas import tpu as pltpu
from jax.experimental.pallas import tpu_sc as plsc

B = 4096
L = 200
D = 64
DP = 128
LANES = 16
NC = 2
NS = 16
NW = NC * NS
BPW = B // NW
NSLOT = 4


def _emb_body(idx_hbm, pos_hbm, tok_hbm, out_hbm, pos_v,
              i0, i1, i2, i3, g0, g1, g2, g3, s0, s1, s2, s3,
              gs0, gs1, gs2, gs3, ss0, ss1, ss2, ss3,
              is0, is1, is2, is3):
    wid = lax.axis_index("s") * NC + lax.axis_index("c")
    w_b = wid * BPW

    idx_b = (i0, i1, i2, i3)
    grows = (g0, g1, g2, g3)
    srows = (s0, s1, s2, s3)
    gsem = (gs0, gs1, gs2, gs3)
    ssem = (ss0, ss1, ss2, ss3)
    isem = (is0, is1, is2, is3)

    def fire_idx(c, b):
        pltpu.async_copy(idx_hbm.at[c, pl.ds(w_b, BPW)], idx_b[b], isem[b])

    def wait_idx(b):
        pltpu.make_async_copy(
            idx_hbm.at[0, pl.ds(0, BPW)], idx_b[b], isem[b]).wait()

    def fire_gather(b):
        pltpu.async_copy(tok_hbm.at[idx_b[b]], grows[b], gsem[b])

    def wait_gather(b):
        pltpu.make_async_copy(tok_hbm.at[idx_b[b]], grows[b], gsem[b]).wait()

    def fire_store(c, b):
        pltpu.async_copy(srows[b], out_hbm.at[c, :, pl.ds(w_b, BPW)], ssem[b])

    def wait_store(b):
        pltpu.make_async_copy(
            srows[b], out_hbm.at[0, :, pl.ds(0, BPW)], ssem[b]).wait()

    pltpu.sync_copy(pos_hbm, pos_v)

    rowv = [lax.iota(jnp.int32, LANES) + j * LANES for j in range(D // LANES)]

    fire_idx(0, 0)
    fire_idx(1, 1)
    fire_idx(2, 2)
    wait_idx(0)
    fire_gather(0)
    wait_idx(1)
    fire_gather(1)

    def outer(t, _):
        for b in range(NSLOT):
            c = t * NSLOT + b
            wait_gather(b)
            gr = grows[b]
            sr = srows[b]
            p = [pos_v[pl.ds(c * D + j * LANES, LANES)]
                 for j in range(D // LANES)]

            @plsc.parallel_loop(0, BPW, unroll=2)
            def _add(tk):
                colv = jnp.full((LANES,), 0, jnp.int32) + tk
                for j in range(D // LANES):
                    x = gr[tk, pl.ds(j * LANES, LANES)] + p[j]
                    plsc.store_scatter(sr, [rowv[j], colv], x)

            fire_store(c, b)

            bn = (b + 2) % NSLOT

            @pl.when(c + 2 < L)
            def _():
                @pl.when(c >= 2)
                def _():
                    wait_store(bn)
                wait_idx(bn)
                fire_gather(bn)

            @pl.when(c + 3 < L)
            def _():
                fire_idx(c + 3, (b + 3) % NSLOT)
        return 0

    lax.fori_loop(0, L // NSLOT, outer, 0, unroll=False)

    for b in range(NSLOT):
        wait_store(b)


@jax.jit
def _emb(idx_t, pos_flat, tok_pad):
    mesh = plsc.VectorSubcoreMesh(core_axis_name="c", subcore_axis_name="s")
    return pl.kernel(
        _emb_body,
        mesh=mesh,
        compiler_params=pltpu.CompilerParams(needs_layout_passes=False),
        out_type=jax.ShapeDtypeStruct((L, D, B), jnp.float32),
        scratch_types=[
            pltpu.VMEM((L * D,), jnp.float32),
        ]
        + [pltpu.VMEM((BPW,), jnp.int32)] * NSLOT
        + [pltpu.VMEM((BPW, DP), jnp.float32)] * NSLOT
        + [pltpu.VMEM((D, BPW), jnp.float32)] * NSLOT
        + [pltpu.SemaphoreType.DMA] * (3 * NSLOT),
    )(idx_t, pos_flat, tok_pad)


def kernel(inputs, token_table, position_table):
    idx_t = jnp.asarray(inputs, jnp.int32).T
    pos_flat = position_table.reshape(L * D)
    tok_pad = jnp.pad(token_table, ((0, 0), (0, DP - D)))
    out_t = _emb(idx_t, pos_flat, tok_pad)
    return jnp.transpose(out_t, (2, 0, 1))


# speedup vs baseline: 2.4095x; 2.4095x over previous
import functools

import jax
import jax.numpy as jnp
from jax import lax
from jax.experimental import pallas as pl
from jax.experimental.pallas import tpu as pltpu
from jax.experimental.pallas import tpu_sc as plsc

B = 4096
L = 200
D = 64
DP = 128
LANES = 16
NC = 2
NS = 16
NW = NC * NS
BPW = B // NW
NSLOT = 4


SKEW = DP + 1


def _emb_body(idx_hbm, pos_hbm, tok_hbm, out_hbm, pos_v, sr_pad,
              i0, i1, i2, i3, g0, g1, g2, g3, s0, s1, s2, s3,
              gs0, gs1, gs2, gs3, ss0, ss1, ss2, ss3,
              is0, is1, is2, is3):
    wid = lax.axis_index("s") * NC + lax.axis_index("c")
    w_b = wid * BPW

    idx_b = (i0, i1, i2, i3)
    grows = (g0, g1, g2, g3)
    srows = (s0, s1, s2, s3)
    gsem = (gs0, gs1, gs2, gs3)
    ssem = (ss0, ss1, ss2, ss3)
    isem = (is0, is1, is2, is3)

    def fire_idx(c, b):
        pltpu.async_copy(idx_hbm.at[c, pl.ds(w_b, BPW)], idx_b[b], isem[b])

    def wait_idx(b):
        pltpu.make_async_copy(
            idx_hbm.at[0, pl.ds(0, BPW)], idx_b[b], isem[b]).wait()

    def fire_gather(b):
        pltpu.async_copy(tok_hbm.at[idx_b[b]], grows[b], gsem[b])

    def wait_gather(b):
        pltpu.make_async_copy(tok_hbm.at[idx_b[b]], grows[b], gsem[b]).wait()

    def fire_store(c, b):
        pltpu.async_copy(srows[b], out_hbm.at[c, :, pl.ds(w_b, BPW)], ssem[b])

    def wait_store(b):
        pltpu.make_async_copy(
            srows[b], out_hbm.at[0, :, pl.ds(0, BPW)], ssem[b]).wait()

    pltpu.sync_copy(pos_hbm, pos_v)

    rowv = [(lax.iota(jnp.int32, LANES) + j * LANES) * SKEW
            for j in range(D // LANES)]

    fire_idx(0, 0)
    fire_idx(1, 1)
    fire_idx(2, 2)
    wait_idx(0)
    fire_gather(0)
    wait_idx(1)
    fire_gather(1)

    def outer(t, _):
        for b in range(NSLOT):
            c = t * NSLOT + b
            wait_gather(b)
            gr = grows[b]
            sr = srows[b]
            p = [pos_v[pl.ds(c * D + j * LANES, LANES)]
                 for j in range(D // LANES)]

            @plsc.parallel_loop(0, BPW, unroll=2)
            def _add(tk):
                for j in range(D // LANES):
                    x = gr[tk, pl.ds(j * LANES, LANES)] + p[j]
                    plsc.store_scatter(sr_pad, [rowv[j] + tk], x)

            @plsc.parallel_loop(0, D, unroll=2)
            def _compact(d):
                for j2 in range(DP // LANES):
                    sr[d, pl.ds(j2 * LANES, LANES)] = (
                        sr_pad[pl.ds(d * SKEW + j2 * LANES, LANES)])

            fire_store(c, b)

            bn = (b + 2) % NSLOT

            @pl.when(c + 2 < L)
            def _():
                @pl.when(c >= 2)
                def _():
                    wait_store(bn)
                wait_idx(bn)
                fire_gather(bn)

            @pl.when(c + 3 < L)
            def _():
                fire_idx(c + 3, (b + 3) % NSLOT)
        return 0

    lax.fori_loop(0, L // NSLOT, outer, 0, unroll=False)

    for b in range(NSLOT):
        wait_store(b)


@jax.jit
def _emb(idx_t, pos_flat, tok_pad):
    mesh = plsc.VectorSubcoreMesh(core_axis_name="c", subcore_axis_name="s")
    return pl.kernel(
        _emb_body,
        mesh=mesh,
        compiler_params=pltpu.CompilerParams(needs_layout_passes=False),
        out_type=jax.ShapeDtypeStruct((L, D, B), jnp.float32),
        scratch_types=[
            pltpu.VMEM((L * D,), jnp.float32),
            pltpu.VMEM((D * SKEW,), jnp.float32),
        ]
        + [pltpu.VMEM((BPW,), jnp.int32)] * NSLOT
        + [pltpu.VMEM((BPW, DP), jnp.float32)] * NSLOT
        + [pltpu.VMEM((D, BPW), jnp.float32)] * NSLOT
        + [pltpu.SemaphoreType.DMA] * (3 * NSLOT),
    )(idx_t, pos_flat, tok_pad)


def kernel(inputs, token_table, position_table):
    idx_t = jnp.asarray(inputs, jnp.int32).T
    pos_flat = position_table.reshape(L * D)
    tok_pad = jnp.pad(token_table, ((0, 0), (0, DP - D)))
    out_t = _emb(idx_t, pos_flat, tok_pad)
    return jnp.transpose(out_t, (2, 0, 1))


# speedup vs baseline: 2.8165x; 1.1689x over previous
import functools

import jax
import jax.numpy as jnp
from jax import lax
from jax.experimental import pallas as pl
from jax.experimental.pallas import tpu as pltpu
from jax.experimental.pallas import tpu_sc as plsc

B = 4096
L = 200
D = 64
DP = 128
LANES = 16
NC = 2
NS = 16
NW = NC * NS
BPW = B // NW
NSLOT = 4
SKEW = DP + 1


def _emb_body(idx_hbm, pos_hbm, tok_hbm, out_hbm, pos_v, sr_pad,
              i0, i1, i2, i3, g0, g1, g2, g3, s0, s1, s2, s3,
              gs0, gs1, gs2, gs3, ss0, ss1, ss2, ss3,
              is0, is1, is2, is3):
    wid = lax.axis_index("s") * NC + lax.axis_index("c")
    w_b = wid * BPW

    idx_b = (i0, i1, i2, i3)
    grows = (g0, g1, g2, g3)
    srows = (s0, s1, s2, s3)
    gsem = (gs0, gs1, gs2, gs3)
    ssem = (ss0, ss1, ss2, ss3)
    isem = (is0, is1, is2, is3)

    def fire_idx(c, b):
        pltpu.async_copy(idx_hbm.at[c, pl.ds(w_b, BPW)], idx_b[b], isem[b])

    def wait_idx(b):
        pltpu.make_async_copy(
            idx_hbm.at[0, pl.ds(0, BPW)], idx_b[b], isem[b]).wait()

    def fire_gather(b):
        pltpu.async_copy(tok_hbm.at[idx_b[b]], grows[b], gsem[b])

    def wait_gather(b):
        pltpu.make_async_copy(tok_hbm.at[idx_b[b]], grows[b], gsem[b]).wait()

    def fire_store(c, b):
        pltpu.async_copy(
            srows[b], out_hbm.at[c, :, pl.ds(wid, 1)], ssem[b])

    def wait_store(b):
        pltpu.make_async_copy(
            srows[b], out_hbm.at[0, :, pl.ds(0, 1)], ssem[b]).wait()

    pltpu.sync_copy(pos_hbm, pos_v)

    rowv = [(lax.iota(jnp.int32, LANES) + j * LANES) * SKEW
            for j in range(D // LANES)]

    fire_idx(0, 0)
    fire_idx(1, 1)
    fire_idx(2, 2)
    wait_idx(0)
    fire_gather(0)
    wait_idx(1)
    fire_gather(1)

    def outer(t, _):
        for b in range(NSLOT):
            c = t * NSLOT + b
            wait_gather(b)
            gr = grows[b]
            sr = srows[b]
            p = [pos_v[pl.ds(c * D + j * LANES, LANES)]
                 for j in range(D // LANES)]

            @plsc.parallel_loop(0, BPW, unroll=2)
            def _add(tk):
                for j in range(D // LANES):
                    x = gr[tk, pl.ds(j * LANES, LANES)] + p[j]
                    plsc.store_scatter(sr_pad, [rowv[j] + tk], x)

            @plsc.parallel_loop(0, D, unroll=2)
            def _compact(d):
                dt = lax.div(d, 8)
                ds = lax.rem(d, 8)
                for j2 in range(DP // LANES):
                    sr[dt, 0, ds, pl.ds(j2 * LANES, LANES)] = (
                        sr_pad[pl.ds(d * SKEW + j2 * LANES, LANES)])

            fire_store(c, b)

            bn = (b + 2) % NSLOT

            @pl.when(c + 2 < L)
            def _():
                @pl.when(c >= 2)
                def _():
                    wait_store(bn)
                wait_idx(bn)
                fire_gather(bn)

            @pl.when(c + 3 < L)
            def _():
                fire_idx(c + 3, (b + 3) % NSLOT)
        return 0

    lax.fori_loop(0, L // NSLOT, outer, 0, unroll=False)

    for b in range(NSLOT):
        wait_store(b)


@jax.jit
def _emb(idx_t, pos_flat, token_table):
    mesh = plsc.VectorSubcoreMesh(core_axis_name="c", subcore_axis_name="s")
    return pl.kernel(
        _emb_body,
        mesh=mesh,
        compiler_params=pltpu.CompilerParams(
            use_tc_tiling_on_sc=False, needs_layout_passes=False),
        out_type=jax.ShapeDtypeStruct((L, D // 8, NW, 8, DP), jnp.float32),
        scratch_types=[
            pltpu.VMEM((L * D,), jnp.float32),
            pltpu.VMEM((D * SKEW,), jnp.float32),
        ]
        + [pltpu.VMEM((BPW,), jnp.int32)] * NSLOT
        + [pltpu.VMEM((BPW, D), jnp.float32)] * NSLOT
        + [pltpu.VMEM((D // 8, 1, 8, DP), jnp.float32)] * NSLOT
        + [pltpu.SemaphoreType.DMA] * (3 * NSLOT),
    )(idx_t, pos_flat, token_table)


def kernel(inputs, token_table, position_table):
    idx_t = jnp.asarray(inputs, jnp.int32).T
    pos_flat = position_table.reshape(L * D)
    out5 = _emb(idx_t, pos_flat, token_table)
    out = jnp.transpose(out5, (2, 4, 0, 1, 3))
    return out.reshape(B, L, D)
